# baseline (speedup 1.0000x reference)
import functools
import math

import jax
import jax.numpy as jnp
from jax import lax
from jax.experimental import pallas as pl
from jax.experimental.pallas import tpu as pltpu
from jax.experimental.pallas import tpu_sc as plsc

N = 10000
D = 128
E = 320000
NP = 10240
BM = 1024
CHUNK = 128
NW = 32
JPT = 80
NCHUNK_P = NW * JPT
EP = NCHUNK_P * CHUNK
HALFJ = JPT // 2
RPW = NP // 16

def _wid():
    core = lax.axis_index("c")
    sub = lax.axis_index("s")
    wid = sub * 2 + core
    return core, sub, wid


def _deg_body(dst_hbm, dega_hbm, degb_hbm, didx_v, ones_v, zbuf_v, deg_sh, sem):
    core, sub, wid = _wid()

    @pl.loop(0, CHUNK // 16)
    def _fill_ones(i):
        ones_v[pl.ds(i * 16, 16)] = jnp.full((16,), 1.0, jnp.float32)

    @pl.loop(0, RPW // 16)
    def _fill_zero(i):
        zbuf_v[pl.ds(i * 16, 16)] = jnp.zeros((16,), jnp.float32)

    pltpu.sync_copy(dst_hbm.at[pl.ds(pl.multiple_of(wid * JPT, JPT), JPT)], didx_v)
    pltpu.sync_copy(zbuf_v, deg_sh.at[pl.ds(pl.multiple_of(sub * RPW, RPW), RPW)])
    plsc.subcore_barrier()

    @pl.loop(0, JPT // 8)
    def _count(it):
        descs = [
            pltpu.async_copy(ones_v, deg_sh.at[didx_v.at[it * 8 + b]], sem, add=True)
            for b in range(8)
        ]
        for desc in descs:
            desc.wait()

    plsc.subcore_barrier()
    off = pl.multiple_of(sub * RPW, RPW)

    @pl.when(core == 0)
    def _():
        pltpu.sync_copy(deg_sh.at[pl.ds(off, RPW)], dega_hbm.at[pl.ds(off, RPW)])

    @pl.when(core == 1)
    def _():
        pltpu.sync_copy(deg_sh.at[pl.ds(off, RPW)], degb_hbm.at[pl.ds(off, RPW)])


def _scatter_body(h_hbm, src_hbm, dst_hbm, pa_hbm, pb_hbm,
                  sidx_v, didx_v, rows0_v, rows1_v, acc_sh,
                  gsem0, gsem1, ssem0, ssem1):
    core, sub, wid = _wid()

    @pl.loop(0, CHUNK)
    def _zrow(r):
        for cc in range(D // 16):
            rows0_v[r, pl.ds(cc * 16, 16)] = jnp.zeros((16,), jnp.float32)

    for k in range(RPW // CHUNK):
        off = pl.multiple_of(sub * RPW + k * CHUNK, CHUNK)
        pltpu.sync_copy(rows0_v, acc_sh.at[pl.ds(off, CHUNK)])

    plsc.subcore_barrier()

    def wait_gather(rows_v, sem):
        pltpu.make_async_copy(h_hbm.at[sidx_v.at[0]], rows_v, sem).wait()

    def wait_scatter(rows_v, sem):
        pltpu.make_async_copy(rows_v, acc_sh.at[didx_v.at[0]], sem).wait()

    for h in range(JPT // HALFJ):
        start = pl.multiple_of(wid * JPT + h * HALFJ, HALFJ)
        pltpu.sync_copy(src_hbm.at[pl.ds(start, HALFJ)], sidx_v)
        pltpu.sync_copy(dst_hbm.at[pl.ds(start, HALFJ)], didx_v)

        pltpu.async_copy(h_hbm.at[sidx_v.at[0]], rows0_v, gsem0)
        pltpu.async_copy(h_hbm.at[sidx_v.at[1]], rows1_v, gsem1)

        @pl.loop(0, HALFJ // 2 - 1)
        def _pipe(it):
            j = it * 2
            wait_gather(rows0_v, gsem0)
            pltpu.async_copy(rows0_v, acc_sh.at[didx_v.at[j]], ssem0, add=True)
            wait_gather(rows1_v, gsem1)
            pltpu.async_copy(rows1_v, acc_sh.at[didx_v.at[j + 1]], ssem1, add=True)
            wait_scatter(rows0_v, ssem0)
            pltpu.async_copy(h_hbm.at[sidx_v.at[j + 2]], rows0_v, gsem0)
            wait_scatter(rows1_v, ssem1)
            pltpu.async_copy(h_hbm.at[sidx_v.at[j + 3]], rows1_v, gsem1)

        wait_gather(rows0_v, gsem0)
        pltpu.async_copy(rows0_v, acc_sh.at[didx_v.at[HALFJ - 2]], ssem0, add=True)
        wait_gather(rows1_v, gsem1)
        pltpu.async_copy(rows1_v, acc_sh.at[didx_v.at[HALFJ - 1]], ssem1, add=True)
        wait_scatter(rows0_v, ssem0)
        wait_scatter(rows1_v, ssem1)

    plsc.subcore_barrier()

    @pl.when(core == 0)
    def _():
        for k in range(RPW // CHUNK):
            off = pl.multiple_of(sub * RPW + k * CHUNK, CHUNK)
            pltpu.sync_copy(acc_sh.at[pl.ds(off, CHUNK)], pa_hbm.at[pl.ds(off, CHUNK)])

    @pl.when(core == 1)
    def _():
        for k in range(RPW // CHUNK):
            off = pl.multiple_of(sub * RPW + k * CHUNK, CHUNK)
            pltpu.sync_copy(acc_sh.at[pl.ds(off, CHUNK)], pb_hbm.at[pl.ds(off, CHUNK)])


@functools.lru_cache(maxsize=None)
def _sc_kernels():
    mesh = plsc.VectorSubcoreMesh(
        core_axis_name="c", subcore_axis_name="s", num_cores=2, num_subcores=16
    )
    deg = pl.kernel(
        _deg_body,
        out_type=(
            jax.ShapeDtypeStruct((NP,), jnp.float32),
            jax.ShapeDtypeStruct((NP,), jnp.float32),
        ),
        mesh=mesh,
        scratch_types=[
            pltpu.VMEM((JPT, CHUNK), jnp.int32),
            pltpu.VMEM((CHUNK,), jnp.float32),
            pltpu.VMEM((RPW,), jnp.float32),
            pltpu.VMEM_SHARED((NP,), jnp.float32),
            pltpu.SemaphoreType.DMA,
        ],
    )
    scatter = pl.kernel(
        _scatter_body,
        out_type=(
            jax.ShapeDtypeStruct((NP, D), jnp.float32),
            jax.ShapeDtypeStruct((NP, D), jnp.float32),
        ),
        mesh=mesh,
        scratch_types=[
            pltpu.VMEM((HALFJ, CHUNK), jnp.int32),
            pltpu.VMEM((HALFJ, CHUNK), jnp.int32),
            pltpu.VMEM((CHUNK, D), jnp.float32),
            pltpu.VMEM((CHUNK, D), jnp.float32),
            pltpu.VMEM_SHARED((NP, D), jnp.float32),
            pltpu.SemaphoreType.DMA,
            pltpu.SemaphoreType.DMA,
            pltpu.SemaphoreType.DMA,
            pltpu.SemaphoreType.DMA,
        ],
    )
    return deg, scatter


def _t1_body(x_ref, w_ref, dega_ref, degb_ref, h_ref, dis_ref):
    deg = dega_ref[...] + degb_ref[...] + 1.0
    dis = lax.rsqrt(deg)
    dis_ref[...] = dis
    h_ref[...] = (
        jnp.dot(x_ref[...], w_ref[...], preferred_element_type=jnp.float32) * dis
    )


_t1 = pl.pallas_call(
    _t1_body,
    grid=(NP // BM,),
    in_specs=[
        pl.BlockSpec((BM, D), lambda i: (i, 0)),
        pl.BlockSpec((D, D), lambda i: (0, 0)),
        pl.BlockSpec((BM, 1), lambda i: (i, 0)),
        pl.BlockSpec((BM, 1), lambda i: (i, 0)),
    ],
    out_specs=[
        pl.BlockSpec((BM, D), lambda i: (i, 0)),
        pl.BlockSpec((BM, 1), lambda i: (i, 0)),
    ],
    out_shape=[
        jax.ShapeDtypeStruct((NP, D), jnp.float32),
        jax.ShapeDtypeStruct((NP, 1), jnp.float32),
    ],
)

_NEG_LOG_OVER_D = -math.log(10000.0) / D


def _t2_body(pa_ref, pb_ref, h1_ref, dis_ref, b1_ref, w2_ref, h2_ref):
    i = pl.program_id(0)
    dis = dis_ref[...]
    agg = dis * (pa_ref[...] + pb_ref[...] + h1_ref[...]) + b1_ref[...]
    pos = (lax.broadcasted_iota(jnp.int32, (BM, D), 0) + i * BM).astype(jnp.float32)
    col = lax.broadcasted_iota(jnp.int32, (BM, D), 1)
    even_exp = ((col // 2) * 2).astype(jnp.float32)
    ang = pos * jnp.exp(even_exp * _NEG_LOG_OVER_D)
    pe = jnp.where(col % 2 == 0, jnp.sin(ang), jnp.cos(ang))
    x1 = jnp.maximum(agg + pe, 0.0)
    h2_ref[...] = (
        jnp.dot(x1, w2_ref[...], preferred_element_type=jnp.float32) * dis
    )


_t2 = pl.pallas_call(
    _t2_body,
    grid=(NP // BM,),
    in_specs=[
        pl.BlockSpec((BM, D), lambda i: (i, 0)),
        pl.BlockSpec((BM, D), lambda i: (i, 0)),
        pl.BlockSpec((BM, D), lambda i: (i, 0)),
        pl.BlockSpec((BM, 1), lambda i: (i, 0)),
        pl.BlockSpec((1, D), lambda i: (0, 0)),
        pl.BlockSpec((D, D), lambda i: (0, 0)),
    ],
    out_specs=pl.BlockSpec((BM, D), lambda i: (i, 0)),
    out_shape=jax.ShapeDtypeStruct((NP, D), jnp.float32),
)


def _t3_body(qa_ref, qb_ref, h2_ref, dis_ref, b2_ref, out_ref):
    out_ref[...] = (
        dis_ref[...] * (qa_ref[...] + qb_ref[...] + h2_ref[...]) + b2_ref[...]
    )


_t3 = pl.pallas_call(
    _t3_body,
    grid=(NP // BM,),
    in_specs=[
        pl.BlockSpec((BM, D), lambda i: (i, 0)),
        pl.BlockSpec((BM, D), lambda i: (i, 0)),
        pl.BlockSpec((BM, D), lambda i: (i, 0)),
        pl.BlockSpec((BM, 1), lambda i: (i, 0)),
        pl.BlockSpec((1, D), lambda i: (0, 0)),
    ],
    out_specs=pl.BlockSpec((BM, D), lambda i: (i, 0)),
    out_shape=jax.ShapeDtypeStruct((NP, D), jnp.float32),
)


def kernel(basic_block, edge_index, W1, b1, W2, b2):
    ei = edge_index.astype(jnp.int32)
    fill = jnp.full((2, EP - E), NP - 1, jnp.int32)
    srcp, dstp = jnp.concatenate([ei, fill], axis=1).reshape(2, NCHUNK_P, CHUNK)
    x = jnp.pad(basic_block, ((0, NP - N), (0, 0)))

    _deg_kernel, _scatter_kernel = _sc_kernels()
    dega, degb = _deg_kernel(dstp)
    h1p, dis = _t1(x, W1, dega.reshape(NP, 1), degb.reshape(NP, 1))
    pa, pb = _scatter_kernel(h1p, srcp, dstp)
    h2p = _t2(pa, pb, h1p, dis, b1.reshape(1, D), W2)
    qa, qb = _scatter_kernel(h2p, srcp, dstp)
    out = _t3(qa, qb, h2p, dis, b2.reshape(1, D))
    return out[:N]


# speedup vs baseline: 2.6781x; 2.6781x over previous
import functools
import math

import jax
import jax.numpy as jnp
from jax import lax
from jax.experimental import pallas as pl
from jax.experimental.pallas import tpu as pltpu
from jax.experimental.pallas import tpu_sc as plsc

N = 10000
D = 128
E = 320000
NP = 10240
BM = 1024
CHUNK = 128
NW = 32
JPT = 80
NCHUNK_P = NW * JPT
EP = NCHUNK_P * CHUNK
HALFJ = JPT // 2
RPW = NP // 16

def _wid():
    core = lax.axis_index("c")
    sub = lax.axis_index("s")
    wid = sub * 2 + core
    return core, sub, wid


def _deg_body(dst_hbm, dega_hbm, degb_hbm, didx_v, ones_v, zbuf_v, deg_sh, sem):
    core, sub, wid = _wid()

    @pl.loop(0, CHUNK // 16)
    def _fill_ones(i):
        ones_v[pl.ds(i * 16, 16)] = jnp.full((16,), 1.0, jnp.float32)

    @pl.loop(0, RPW // 16)
    def _fill_zero(i):
        zbuf_v[pl.ds(i * 16, 16)] = jnp.zeros((16,), jnp.float32)

    pltpu.sync_copy(dst_hbm.at[pl.ds(pl.multiple_of(wid * JPT, JPT), JPT)], didx_v)
    pltpu.sync_copy(zbuf_v, deg_sh.at[pl.ds(pl.multiple_of(sub * RPW, RPW), RPW)])
    plsc.subcore_barrier()

    @pl.loop(0, JPT // 8)
    def _count(it):
        descs = [
            pltpu.async_copy(ones_v, deg_sh.at[didx_v.at[it * 8 + b]], sem, add=True)
            for b in range(8)
        ]
        for desc in descs:
            desc.wait()

    plsc.subcore_barrier()
    off = pl.multiple_of(sub * RPW, RPW)

    @pl.when(core == 0)
    def _():
        pltpu.sync_copy(deg_sh.at[pl.ds(off, RPW)], dega_hbm.at[pl.ds(off, RPW)])

    @pl.when(core == 1)
    def _():
        pltpu.sync_copy(deg_sh.at[pl.ds(off, RPW)], degb_hbm.at[pl.ds(off, RPW)])


def _scatter_body(h_hbm, src_hbm, dst_hbm, pa_hbm, pb_hbm,
                  sidx_v, didx_v, rows0_v, rows1_v, acc_sh,
                  gsem0, gsem1, ssem0, ssem1):
    core, sub, wid = _wid()

    @pl.loop(0, CHUNK)
    def _zrow(r):
        for cc in range(D // 16):
            rows0_v[r, pl.ds(cc * 16, 16)] = jnp.zeros((16,), jnp.float32)

    for k in range(RPW // CHUNK):
        off = pl.multiple_of(sub * RPW + k * CHUNK, CHUNK)
        pltpu.sync_copy(rows0_v, acc_sh.at[pl.ds(off, CHUNK)])

    plsc.subcore_barrier()

    def wait_gather(rows_v, sem):
        pltpu.make_async_copy(h_hbm.at[sidx_v.at[0]], rows_v, sem).wait()

    def wait_scatter(rows_v, sem):
        pltpu.make_async_copy(rows_v, acc_sh.at[didx_v.at[0]], sem).wait()

    for h in range(JPT // HALFJ):
        start = pl.multiple_of(wid * JPT + h * HALFJ, HALFJ)
        pltpu.sync_copy(src_hbm.at[pl.ds(start, HALFJ)], sidx_v)
        pltpu.sync_copy(dst_hbm.at[pl.ds(start, HALFJ)], didx_v)

        pltpu.async_copy(h_hbm.at[sidx_v.at[0]], rows0_v, gsem0)
        pltpu.async_copy(h_hbm.at[sidx_v.at[1]], rows1_v, gsem1)

        @pl.loop(0, HALFJ // 2 - 1)
        def _pipe(it):
            j = it * 2
            wait_gather(rows0_v, gsem0)
            pltpu.async_copy(rows0_v, acc_sh.at[didx_v.at[j]], ssem0, add=True)
            wait_gather(rows1_v, gsem1)
            pltpu.async_copy(rows1_v, acc_sh.at[didx_v.at[j + 1]], ssem1, add=True)
            wait_scatter(rows0_v, ssem0)
            pltpu.async_copy(h_hbm.at[sidx_v.at[j + 2]], rows0_v, gsem0)
            wait_scatter(rows1_v, ssem1)
            pltpu.async_copy(h_hbm.at[sidx_v.at[j + 3]], rows1_v, gsem1)

        wait_gather(rows0_v, gsem0)
        pltpu.async_copy(rows0_v, acc_sh.at[didx_v.at[HALFJ - 2]], ssem0, add=True)
        wait_gather(rows1_v, gsem1)
        pltpu.async_copy(rows1_v, acc_sh.at[didx_v.at[HALFJ - 1]], ssem1, add=True)
        wait_scatter(rows0_v, ssem0)
        wait_scatter(rows1_v, ssem1)

    plsc.subcore_barrier()

    @pl.when(core == 0)
    def _():
        for k in range(RPW // CHUNK):
            off = pl.multiple_of(sub * RPW + k * CHUNK, CHUNK)
            pltpu.sync_copy(acc_sh.at[pl.ds(off, CHUNK)], pa_hbm.at[pl.ds(off, CHUNK)])

    @pl.when(core == 1)
    def _():
        for k in range(RPW // CHUNK):
            off = pl.multiple_of(sub * RPW + k * CHUNK, CHUNK)
            pltpu.sync_copy(acc_sh.at[pl.ds(off, CHUNK)], pb_hbm.at[pl.ds(off, CHUNK)])


@functools.lru_cache(maxsize=None)
def _sc_kernels():
    mesh = plsc.VectorSubcoreMesh(
        core_axis_name="c", subcore_axis_name="s", num_cores=2, num_subcores=16
    )
    deg = pl.kernel(
        _deg_body,
        out_type=(
            jax.ShapeDtypeStruct((NP,), jnp.float32),
            jax.ShapeDtypeStruct((NP,), jnp.float32),
        ),
        mesh=mesh,
        scratch_types=[
            pltpu.VMEM((JPT, CHUNK), jnp.int32),
            pltpu.VMEM((CHUNK,), jnp.float32),
            pltpu.VMEM((RPW,), jnp.float32),
            pltpu.VMEM_SHARED((NP,), jnp.float32),
            pltpu.SemaphoreType.DMA,
        ],
    )
    scatter = pl.kernel(
        _scatter_body,
        out_type=(
            jax.ShapeDtypeStruct((NP, D), jnp.float32),
            jax.ShapeDtypeStruct((NP, D), jnp.float32),
        ),
        mesh=mesh,
        scratch_types=[
            pltpu.VMEM((HALFJ, CHUNK), jnp.int32),
            pltpu.VMEM((HALFJ, CHUNK), jnp.int32),
            pltpu.VMEM((CHUNK, D), jnp.float32),
            pltpu.VMEM((CHUNK, D), jnp.float32),
            pltpu.VMEM_SHARED((NP, D), jnp.float32),
            pltpu.SemaphoreType.DMA,
            pltpu.SemaphoreType.DMA,
            pltpu.SemaphoreType.DMA,
            pltpu.SemaphoreType.DMA,
        ],
    )
    return deg, scatter


def _t1_body(x_ref, w_ref, dega_ref, degb_ref, h_ref, dis_ref):
    deg = dega_ref[...] + degb_ref[...] + 1.0
    dis = lax.rsqrt(deg)
    dis_ref[...] = dis
    h_ref[...] = (
        jnp.dot(x_ref[...], w_ref[...], preferred_element_type=jnp.float32) * dis
    )


_t1 = pl.pallas_call(
    _t1_body,
    grid=(NP // BM,),
    in_specs=[
        pl.BlockSpec((BM, D), lambda i: (i, 0)),
        pl.BlockSpec((D, D), lambda i: (0, 0)),
        pl.BlockSpec((BM, 1), lambda i: (i, 0)),
        pl.BlockSpec((BM, 1), lambda i: (i, 0)),
    ],
    out_specs=[
        pl.BlockSpec((BM, D), lambda i: (i, 0)),
        pl.BlockSpec((BM, 1), lambda i: (i, 0)),
    ],
    out_shape=[
        jax.ShapeDtypeStruct((NP, D), jnp.float32),
        jax.ShapeDtypeStruct((NP, 1), jnp.float32),
    ],
)

_NEG_LOG_OVER_D = -math.log(10000.0) / D


def _t2_body(pa_ref, pb_ref, h1_ref, dis_ref, b1_ref, w2_ref, h2_ref):
    i = pl.program_id(0)
    dis = dis_ref[...]
    agg = dis * (pa_ref[...] + pb_ref[...] + h1_ref[...]) + b1_ref[...]
    pos = (lax.broadcasted_iota(jnp.int32, (BM, D), 0) + i * BM).astype(jnp.float32)
    col = lax.broadcasted_iota(jnp.int32, (BM, D), 1)
    even_exp = ((col // 2) * 2).astype(jnp.float32)
    ang = pos * jnp.exp(even_exp * _NEG_LOG_OVER_D)
    pe = jnp.where(col % 2 == 0, jnp.sin(ang), jnp.cos(ang))
    x1 = jnp.maximum(agg + pe, 0.0)
    h2_ref[...] = (
        jnp.dot(x1, w2_ref[...], preferred_element_type=jnp.float32) * dis
    )


_t2 = pl.pallas_call(
    _t2_body,
    grid=(NP // BM,),
    in_specs=[
        pl.BlockSpec((BM, D), lambda i: (i, 0)),
        pl.BlockSpec((BM, D), lambda i: (i, 0)),
        pl.BlockSpec((BM, D), lambda i: (i, 0)),
        pl.BlockSpec((BM, 1), lambda i: (i, 0)),
        pl.BlockSpec((1, D), lambda i: (0, 0)),
        pl.BlockSpec((D, D), lambda i: (0, 0)),
    ],
    out_specs=pl.BlockSpec((BM, D), lambda i: (i, 0)),
    out_shape=jax.ShapeDtypeStruct((NP, D), jnp.float32),
)


def _t3_body(qa_ref, qb_ref, h2_ref, dis_ref, b2_ref, out_ref):
    out_ref[...] = (
        dis_ref[...] * (qa_ref[...] + qb_ref[...] + h2_ref[...]) + b2_ref[...]
    )


_t3 = pl.pallas_call(
    _t3_body,
    grid=(NP // BM,),
    in_specs=[
        pl.BlockSpec((BM, D), lambda i: (i, 0)),
        pl.BlockSpec((BM, D), lambda i: (i, 0)),
        pl.BlockSpec((BM, D), lambda i: (i, 0)),
        pl.BlockSpec((BM, 1), lambda i: (i, 0)),
        pl.BlockSpec((1, D), lambda i: (0, 0)),
    ],
    out_specs=pl.BlockSpec((BM, D), lambda i: (i, 0)),
    out_shape=jax.ShapeDtypeStruct((NP, D), jnp.float32),
)


def kernel(basic_block, edge_index, W1, b1, W2, b2):
    ei = edge_index.astype(jnp.int32)
    fill = jnp.broadcast_to(N + jnp.arange(EP - E, dtype=jnp.int32) % (NP - N),
                            (2, EP - E))
    srcp, dstp = jnp.concatenate([ei, fill], axis=1).reshape(2, NCHUNK_P, CHUNK)
    x = jnp.pad(basic_block, ((0, NP - N), (0, 0)))

    _deg_kernel, _scatter_kernel = _sc_kernels()
    dega, degb = _deg_kernel(dstp)
    h1p, dis = _t1(x, W1, dega.reshape(NP, 1), degb.reshape(NP, 1))
    pa, pb = _scatter_kernel(h1p, srcp, dstp)
    h2p = _t2(pa, pb, h1p, dis, b1.reshape(1, D), W2)
    qa, qb = _scatter_kernel(h2p, srcp, dstp)
    out = _t3(qa, qb, h2p, dis, b2.reshape(1, D))
    return out[:N]


# speedup vs baseline: 2.8256x; 1.0551x over previous
import functools
import math

import numpy as np

import jax
import jax.numpy as jnp
from jax import lax
from jax.experimental import pallas as pl
from jax.experimental.pallas import tpu as pltpu
from jax.experimental.pallas import tpu_sc as plsc

N = 10000
D = 128
E = 320000
NP = 10240
BM = 1000
CHUNK = 128
NW = 32
JPT = 80
NCHUNK_P = NW * JPT
EP = NCHUNK_P * CHUNK
HALFJ = JPT // 2
RPW = NP // 16

def _wid():
    core = lax.axis_index("c")
    sub = lax.axis_index("s")
    wid = sub * 2 + core
    return core, sub, wid


def _deg_body(dst_hbm, dega_hbm, degb_hbm, didx_v, ones_v, zbuf_v, deg_sh, sem):
    core, sub, wid = _wid()

    @pl.loop(0, CHUNK // 16)
    def _fill_ones(i):
        ones_v[pl.ds(i * 16, 16)] = jnp.full((16,), 1.0, jnp.float32)

    @pl.loop(0, RPW // 16)
    def _fill_zero(i):
        zbuf_v[pl.ds(i * 16, 16)] = jnp.zeros((16,), jnp.float32)

    pltpu.sync_copy(dst_hbm.at[pl.ds(pl.multiple_of(wid * JPT, JPT), JPT)], didx_v)
    pltpu.sync_copy(zbuf_v, deg_sh.at[pl.ds(pl.multiple_of(sub * RPW, RPW), RPW)])
    plsc.subcore_barrier()

    @pl.loop(0, JPT // 8)
    def _count(it):
        descs = [
            pltpu.async_copy(ones_v, deg_sh.at[didx_v.at[it * 8 + b]], sem, add=True)
            for b in range(8)
        ]
        for desc in descs:
            desc.wait()

    plsc.subcore_barrier()
    off = pl.multiple_of(sub * RPW, RPW)

    @pl.when(core == 0)
    def _():
        pltpu.sync_copy(deg_sh.at[pl.ds(off, RPW)], dega_hbm.at[pl.ds(off, RPW)])

    @pl.when(core == 1)
    def _():
        pltpu.sync_copy(deg_sh.at[pl.ds(off, RPW)], degb_hbm.at[pl.ds(off, RPW)])


def _scatter_body(h_hbm, src_hbm, dst_hbm, pa_hbm, pb_hbm,
                  sidx_v, didx_v, rows0_v, rows1_v, acc_sh,
                  gsem0, gsem1, ssem0, ssem1):
    core, sub, wid = _wid()

    @pl.loop(0, CHUNK)
    def _zrow(r):
        for cc in range(D // 16):
            rows0_v[r, pl.ds(cc * 16, 16)] = jnp.zeros((16,), jnp.float32)

    for k in range(RPW // CHUNK):
        off = pl.multiple_of(sub * RPW + k * CHUNK, CHUNK)
        pltpu.sync_copy(rows0_v, acc_sh.at[pl.ds(off, CHUNK)])

    plsc.subcore_barrier()

    def wait_gather(rows_v, sem):
        pltpu.make_async_copy(h_hbm.at[sidx_v.at[0]], rows_v, sem).wait()

    def wait_scatter(rows_v, sem):
        pltpu.make_async_copy(rows_v, acc_sh.at[didx_v.at[0]], sem).wait()

    for h in range(JPT // HALFJ):
        start = pl.multiple_of(wid * JPT + h * HALFJ, HALFJ)
        pltpu.sync_copy(src_hbm.at[pl.ds(start, HALFJ)], sidx_v)
        pltpu.sync_copy(dst_hbm.at[pl.ds(start, HALFJ)], didx_v)

        pltpu.async_copy(h_hbm.at[sidx_v.at[0]], rows0_v, gsem0)
        pltpu.async_copy(h_hbm.at[sidx_v.at[1]], rows1_v, gsem1)

        @pl.loop(0, HALFJ // 2 - 1)
        def _pipe(it):
            j = it * 2
            wait_gather(rows0_v, gsem0)
            pltpu.async_copy(rows0_v, acc_sh.at[didx_v.at[j]], ssem0, add=True)
            wait_gather(rows1_v, gsem1)
            pltpu.async_copy(rows1_v, acc_sh.at[didx_v.at[j + 1]], ssem1, add=True)
            wait_scatter(rows0_v, ssem0)
            pltpu.async_copy(h_hbm.at[sidx_v.at[j + 2]], rows0_v, gsem0)
            wait_scatter(rows1_v, ssem1)
            pltpu.async_copy(h_hbm.at[sidx_v.at[j + 3]], rows1_v, gsem1)

        wait_gather(rows0_v, gsem0)
        pltpu.async_copy(rows0_v, acc_sh.at[didx_v.at[HALFJ - 2]], ssem0, add=True)
        wait_gather(rows1_v, gsem1)
        pltpu.async_copy(rows1_v, acc_sh.at[didx_v.at[HALFJ - 1]], ssem1, add=True)
        wait_scatter(rows0_v, ssem0)
        wait_scatter(rows1_v, ssem1)

    plsc.subcore_barrier()

    @pl.when(core == 0)
    def _():
        for k in range(RPW // CHUNK):
            off = pl.multiple_of(sub * RPW + k * CHUNK, CHUNK)
            pltpu.sync_copy(acc_sh.at[pl.ds(off, CHUNK)], pa_hbm.at[pl.ds(off, CHUNK)])

    @pl.when(core == 1)
    def _():
        for k in range(RPW // CHUNK):
            off = pl.multiple_of(sub * RPW + k * CHUNK, CHUNK)
            pltpu.sync_copy(acc_sh.at[pl.ds(off, CHUNK)], pb_hbm.at[pl.ds(off, CHUNK)])


@functools.lru_cache(maxsize=None)
def _sc_kernels():
    mesh = plsc.VectorSubcoreMesh(
        core_axis_name="c", subcore_axis_name="s", num_cores=2, num_subcores=16
    )
    deg = pl.kernel(
        _deg_body,
        out_type=(
            jax.ShapeDtypeStruct((NP,), jnp.float32),
            jax.ShapeDtypeStruct((NP,), jnp.float32),
        ),
        mesh=mesh,
        scratch_types=[
            pltpu.VMEM((JPT, CHUNK), jnp.int32),
            pltpu.VMEM((CHUNK,), jnp.float32),
            pltpu.VMEM((RPW,), jnp.float32),
            pltpu.VMEM_SHARED((NP,), jnp.float32),
            pltpu.SemaphoreType.DMA,
        ],
    )
    scatter = pl.kernel(
        _scatter_body,
        out_type=(
            jax.ShapeDtypeStruct((NP, D), jnp.float32),
            jax.ShapeDtypeStruct((NP, D), jnp.float32),
        ),
        mesh=mesh,
        scratch_types=[
            pltpu.VMEM((HALFJ, CHUNK), jnp.int32),
            pltpu.VMEM((HALFJ, CHUNK), jnp.int32),
            pltpu.VMEM((CHUNK, D), jnp.float32),
            pltpu.VMEM((CHUNK, D), jnp.float32),
            pltpu.VMEM_SHARED((NP, D), jnp.float32),
            pltpu.SemaphoreType.DMA,
            pltpu.SemaphoreType.DMA,
            pltpu.SemaphoreType.DMA,
            pltpu.SemaphoreType.DMA,
        ],
    )
    return deg, scatter


def _t1_body(x_ref, w_ref, dega_ref, degb_ref, h_ref, dis_ref):
    deg = dega_ref[...] + degb_ref[...] + 1.0
    dis = lax.rsqrt(deg)
    dis_ref[...] = dis
    h_ref[...] = (
        jnp.dot(x_ref[...], w_ref[...], preferred_element_type=jnp.float32) * dis
    )


_t1 = pl.pallas_call(
    _t1_body,
    grid=(N // BM,),
    in_specs=[
        pl.BlockSpec((BM, D), lambda i: (i, 0)),
        pl.BlockSpec((D, D), lambda i: (0, 0)),
        pl.BlockSpec((BM, 1), lambda i: (i, 0)),
        pl.BlockSpec((BM, 1), lambda i: (i, 0)),
    ],
    out_specs=[
        pl.BlockSpec((BM, D), lambda i: (i, 0)),
        pl.BlockSpec((BM, 1), lambda i: (i, 0)),
    ],
    out_shape=[
        jax.ShapeDtypeStruct((N, D), jnp.float32),
        jax.ShapeDtypeStruct((N, 1), jnp.float32),
    ],
)


def _pe_table():
    pos = np.arange(N, dtype=np.float32)[:, None]
    div = np.exp(np.arange(0, D, 2, dtype=np.float32) * np.float32(-math.log(10000.0) / D))
    pe = np.zeros((N, D), dtype=np.float32)
    pe[:, 0::2] = np.sin(pos * div, dtype=np.float32)
    pe[:, 1::2] = np.cos(pos * div, dtype=np.float32)
    return pe


_PE = _pe_table()


def _t2_body(pa_ref, pb_ref, h1_ref, dis_ref, pe_ref, b1_ref, w2_ref, h2_ref):
    dis = dis_ref[...]
    agg = dis * (pa_ref[...] + pb_ref[...] + h1_ref[...]) + b1_ref[...]
    x1 = jnp.maximum(agg + pe_ref[...], 0.0)
    h2_ref[...] = (
        jnp.dot(x1, w2_ref[...], preferred_element_type=jnp.float32) * dis
    )


_t2 = pl.pallas_call(
    _t2_body,
    grid=(N // BM,),
    in_specs=[
        pl.BlockSpec((BM, D), lambda i: (i, 0)),
        pl.BlockSpec((BM, D), lambda i: (i, 0)),
        pl.BlockSpec((BM, D), lambda i: (i, 0)),
        pl.BlockSpec((BM, 1), lambda i: (i, 0)),
        pl.BlockSpec((BM, D), lambda i: (i, 0)),
        pl.BlockSpec((1, D), lambda i: (0, 0)),
        pl.BlockSpec((D, D), lambda i: (0, 0)),
    ],
    out_specs=pl.BlockSpec((BM, D), lambda i: (i, 0)),
    out_shape=jax.ShapeDtypeStruct((N, D), jnp.float32),
)


def _t3_body(qa_ref, qb_ref, h2_ref, dis_ref, b2_ref, out_ref):
    out_ref[...] = (
        dis_ref[...] * (qa_ref[...] + qb_ref[...] + h2_ref[...]) + b2_ref[...]
    )


_t3 = pl.pallas_call(
    _t3_body,
    grid=(N // BM,),
    in_specs=[
        pl.BlockSpec((BM, D), lambda i: (i, 0)),
        pl.BlockSpec((BM, D), lambda i: (i, 0)),
        pl.BlockSpec((BM, D), lambda i: (i, 0)),
        pl.BlockSpec((BM, 1), lambda i: (i, 0)),
        pl.BlockSpec((1, D), lambda i: (0, 0)),
    ],
    out_specs=pl.BlockSpec((BM, D), lambda i: (i, 0)),
    out_shape=jax.ShapeDtypeStruct((N, D), jnp.float32),
)


def kernel(basic_block, edge_index, W1, b1, W2, b2):
    ei = edge_index.astype(jnp.int32)
    cyc = jnp.arange(EP - E, dtype=jnp.int32) % (NP - N)
    fill = jnp.stack([cyc, N + cyc])
    srcp, dstp = jnp.concatenate([ei, fill], axis=1).reshape(2, NCHUNK_P, CHUNK)

    _deg_kernel, _scatter_kernel = _sc_kernels()
    dega, degb = _deg_kernel(dstp)
    h1p, dis = _t1(basic_block, W1, dega.reshape(NP, 1), degb.reshape(NP, 1))
    pa, pb = _scatter_kernel(h1p, srcp, dstp)
    h2p = _t2(pa, pb, h1p, dis, jnp.asarray(_PE), b1.reshape(1, D), W2)
    qa, qb = _scatter_kernel(h2p, srcp, dstp)
    return _t3(qa, qb, h2p, dis, b2.reshape(1, D))


# speedup vs baseline: 3.1254x; 1.1061x over previous
import functools
import math

import numpy as np

import jax
import jax.numpy as jnp
from jax import lax
from jax.experimental import pallas as pl
from jax.experimental.pallas import tpu as pltpu
from jax.experimental.pallas import tpu_sc as plsc

N = 10000
D = 128
E = 320000
NP = 10240
BM = 1000
CHUNK = 64
NW = 32
JPT = 160
NCHUNK_P = NW * JPT
EP = NCHUNK_P * CHUNK
PIECE = JPT // 4
NBUF = 4
RPW = NP // 16

def _wid():
    core = lax.axis_index("c")
    sub = lax.axis_index("s")
    wid = sub * 2 + core
    return core, sub, wid


def _deg_body(dst_hbm, dega_hbm, degb_hbm, didx_v, ones_v, zbuf_v, deg_sh, sem):
    core, sub, wid = _wid()

    @pl.loop(0, CHUNK // 16)
    def _fill_ones(i):
        ones_v[pl.ds(i * 16, 16)] = jnp.full((16,), 1.0, jnp.float32)

    @pl.loop(0, RPW // 16)
    def _fill_zero(i):
        zbuf_v[pl.ds(i * 16, 16)] = jnp.zeros((16,), jnp.float32)

    pltpu.sync_copy(dst_hbm.at[pl.ds(pl.multiple_of(wid * JPT, JPT), JPT)], didx_v)
    pltpu.sync_copy(zbuf_v, deg_sh.at[pl.ds(pl.multiple_of(sub * RPW, RPW), RPW)])
    plsc.subcore_barrier()

    @pl.loop(0, JPT // 8)
    def _count(it):
        descs = [
            pltpu.async_copy(ones_v, deg_sh.at[didx_v.at[it * 8 + b]], sem, add=True)
            for b in range(8)
        ]
        for desc in descs:
            desc.wait()

    plsc.subcore_barrier()
    off = pl.multiple_of(sub * RPW, RPW)

    @pl.when(core == 0)
    def _():
        pltpu.sync_copy(deg_sh.at[pl.ds(off, RPW)], dega_hbm.at[pl.ds(off, RPW)])

    @pl.when(core == 1)
    def _():
        pltpu.sync_copy(deg_sh.at[pl.ds(off, RPW)], degb_hbm.at[pl.ds(off, RPW)])


def _scatter_body(h_hbm, src_hbm, dst_hbm, pa_hbm, pb_hbm,
                  sidx_v, didx_v, rows0_v, rows1_v, rows2_v, rows3_v, acc_sh,
                  gsem0, gsem1, gsem2, gsem3, ssem0, ssem1, ssem2, ssem3):
    core, sub, wid = _wid()
    rows = (rows0_v, rows1_v, rows2_v, rows3_v)
    gsem = (gsem0, gsem1, gsem2, gsem3)
    ssem = (ssem0, ssem1, ssem2, ssem3)

    @pl.loop(0, CHUNK)
    def _zrow(r):
        for cc in range(D // 16):
            rows0_v[r, pl.ds(cc * 16, 16)] = jnp.zeros((16,), jnp.float32)

    for k in range(RPW // CHUNK):
        off = pl.multiple_of(sub * RPW + k * CHUNK, CHUNK)
        pltpu.sync_copy(rows0_v, acc_sh.at[pl.ds(off, CHUNK)])

    plsc.subcore_barrier()

    def wait_gather(rows_v, sem):
        pltpu.make_async_copy(h_hbm.at[sidx_v.at[0]], rows_v, sem).wait()

    def wait_scatter(rows_v, sem):
        pltpu.make_async_copy(rows_v, acc_sh.at[didx_v.at[0]], sem).wait()

    for h in range(JPT // PIECE):
        start = pl.multiple_of(wid * JPT + h * PIECE, PIECE)
        pltpu.sync_copy(src_hbm.at[pl.ds(start, PIECE)], sidx_v)
        pltpu.sync_copy(dst_hbm.at[pl.ds(start, PIECE)], didx_v)

        for b in range(NBUF):
            pltpu.async_copy(h_hbm.at[sidx_v.at[b]], rows[b], gsem[b])

        @pl.loop(0, PIECE // NBUF - 1)
        def _pipe(it):
            j = it * NBUF
            for b in range(NBUF):
                wait_gather(rows[b], gsem[b])
                pltpu.async_copy(rows[b], acc_sh.at[didx_v.at[j + b]], ssem[b],
                                 add=True)
            for b in range(NBUF):
                wait_scatter(rows[b], ssem[b])
                pltpu.async_copy(h_hbm.at[sidx_v.at[j + NBUF + b]], rows[b],
                                 gsem[b])

        for b in range(NBUF):
            wait_gather(rows[b], gsem[b])
            pltpu.async_copy(rows[b], acc_sh.at[didx_v.at[PIECE - NBUF + b]],
                             ssem[b], add=True)
        for b in range(NBUF):
            wait_scatter(rows[b], ssem[b])

    plsc.subcore_barrier()

    @pl.when(core == 0)
    def _():
        for k in range(RPW // CHUNK):
            off = pl.multiple_of(sub * RPW + k * CHUNK, CHUNK)
            pltpu.sync_copy(acc_sh.at[pl.ds(off, CHUNK)], pa_hbm.at[pl.ds(off, CHUNK)])

    @pl.when(core == 1)
    def _():
        for k in range(RPW // CHUNK):
            off = pl.multiple_of(sub * RPW + k * CHUNK, CHUNK)
            pltpu.sync_copy(acc_sh.at[pl.ds(off, CHUNK)], pb_hbm.at[pl.ds(off, CHUNK)])


@functools.lru_cache(maxsize=None)
def _sc_kernels():
    mesh = plsc.VectorSubcoreMesh(
        core_axis_name="c", subcore_axis_name="s", num_cores=2, num_subcores=16
    )
    deg = pl.kernel(
        _deg_body,
        out_type=(
            jax.ShapeDtypeStruct((NP,), jnp.float32),
            jax.ShapeDtypeStruct((NP,), jnp.float32),
        ),
        mesh=mesh,
        scratch_types=[
            pltpu.VMEM((JPT, CHUNK), jnp.int32),
            pltpu.VMEM((CHUNK,), jnp.float32),
            pltpu.VMEM((RPW,), jnp.float32),
            pltpu.VMEM_SHARED((NP,), jnp.float32),
            pltpu.SemaphoreType.DMA,
        ],
    )
    scatter = pl.kernel(
        _scatter_body,
        out_type=(
            jax.ShapeDtypeStruct((NP, D), jnp.float32),
            jax.ShapeDtypeStruct((NP, D), jnp.float32),
        ),
        mesh=mesh,
        scratch_types=[
            pltpu.VMEM((PIECE, CHUNK), jnp.int32),
            pltpu.VMEM((PIECE, CHUNK), jnp.int32),
            pltpu.VMEM((CHUNK, D), jnp.float32),
            pltpu.VMEM((CHUNK, D), jnp.float32),
            pltpu.VMEM((CHUNK, D), jnp.float32),
            pltpu.VMEM((CHUNK, D), jnp.float32),
            pltpu.VMEM_SHARED((NP, D), jnp.float32),
            pltpu.SemaphoreType.DMA,
            pltpu.SemaphoreType.DMA,
            pltpu.SemaphoreType.DMA,
            pltpu.SemaphoreType.DMA,
            pltpu.SemaphoreType.DMA,
            pltpu.SemaphoreType.DMA,
            pltpu.SemaphoreType.DMA,
            pltpu.SemaphoreType.DMA,
        ],
    )
    return deg, scatter


def _t1_body(x_ref, w_ref, dega_ref, degb_ref, h_ref, dis_ref):
    deg = dega_ref[...] + degb_ref[...] + 1.0
    dis = lax.rsqrt(deg)
    dis_ref[...] = dis
    h_ref[...] = (
        jnp.dot(x_ref[...], w_ref[...], preferred_element_type=jnp.float32) * dis
    )


_t1 = pl.pallas_call(
    _t1_body,
    grid=(N // BM,),
    in_specs=[
        pl.BlockSpec((BM, D), lambda i: (i, 0)),
        pl.BlockSpec((D, D), lambda i: (0, 0)),
        pl.BlockSpec((BM, 1), lambda i: (i, 0)),
        pl.BlockSpec((BM, 1), lambda i: (i, 0)),
    ],
    out_specs=[
        pl.BlockSpec((BM, D), lambda i: (i, 0)),
        pl.BlockSpec((BM, 1), lambda i: (i, 0)),
    ],
    out_shape=[
        jax.ShapeDtypeStruct((N, D), jnp.float32),
        jax.ShapeDtypeStruct((N, 1), jnp.float32),
    ],
)


def _pe_table():
    pos = np.arange(N, dtype=np.float32)[:, None]
    div = np.exp(np.arange(0, D, 2, dtype=np.float32) * np.float32(-math.log(10000.0) / D))
    pe = np.zeros((N, D), dtype=np.float32)
    pe[:, 0::2] = np.sin(pos * div, dtype=np.float32)
    pe[:, 1::2] = np.cos(pos * div, dtype=np.float32)
    return pe


_PE = _pe_table()


def _t2_body(pa_ref, pb_ref, h1_ref, dis_ref, pe_ref, b1_ref, w2_ref, h2_ref):
    dis = dis_ref[...]
    agg = dis * (pa_ref[...] + pb_ref[...] + h1_ref[...]) + b1_ref[...]
    x1 = jnp.maximum(agg + pe_ref[...], 0.0)
    h2_ref[...] = (
        jnp.dot(x1, w2_ref[...], preferred_element_type=jnp.float32) * dis
    )


_t2 = pl.pallas_call(
    _t2_body,
    grid=(N // BM,),
    in_specs=[
        pl.BlockSpec((BM, D), lambda i: (i, 0)),
        pl.BlockSpec((BM, D), lambda i: (i, 0)),
        pl.BlockSpec((BM, D), lambda i: (i, 0)),
        pl.BlockSpec((BM, 1), lambda i: (i, 0)),
        pl.BlockSpec((BM, D), lambda i: (i, 0)),
        pl.BlockSpec((1, D), lambda i: (0, 0)),
        pl.BlockSpec((D, D), lambda i: (0, 0)),
    ],
    out_specs=pl.BlockSpec((BM, D), lambda i: (i, 0)),
    out_shape=jax.ShapeDtypeStruct((N, D), jnp.float32),
)


def _t3_body(qa_ref, qb_ref, h2_ref, dis_ref, b2_ref, out_ref):
    out_ref[...] = (
        dis_ref[...] * (qa_ref[...] + qb_ref[...] + h2_ref[...]) + b2_ref[...]
    )


_t3 = pl.pallas_call(
    _t3_body,
    grid=(N // BM,),
    in_specs=[
        pl.BlockSpec((BM, D), lambda i: (i, 0)),
        pl.BlockSpec((BM, D), lambda i: (i, 0)),
        pl.BlockSpec((BM, D), lambda i: (i, 0)),
        pl.BlockSpec((BM, 1), lambda i: (i, 0)),
        pl.BlockSpec((1, D), lambda i: (0, 0)),
    ],
    out_specs=pl.BlockSpec((BM, D), lambda i: (i, 0)),
    out_shape=jax.ShapeDtypeStruct((N, D), jnp.float32),
)


def kernel(basic_block, edge_index, W1, b1, W2, b2):
    ei = edge_index.astype(jnp.int32)
    cyc = jnp.arange(EP - E, dtype=jnp.int32) % (NP - N)
    fill = jnp.stack([cyc, N + cyc])
    srcp, dstp = jnp.concatenate([ei, fill], axis=1).reshape(2, NCHUNK_P, CHUNK)

    _deg_kernel, _scatter_kernel = _sc_kernels()
    dega, degb = _deg_kernel(dstp)
    h1p, dis = _t1(basic_block, W1, dega.reshape(NP, 1), degb.reshape(NP, 1))
    pa, pb = _scatter_kernel(h1p, srcp, dstp)
    h2p = _t2(pa, pb, h1p, dis, jnp.asarray(_PE), b1.reshape(1, D), W2)
    qa, qb = _scatter_kernel(h2p, srcp, dstp)
    return _t3(qa, qb, h2p, dis, b2.reshape(1, D))


# speedup vs baseline: 3.1383x; 1.0041x over previous
import functools
import math

import numpy as np

import jax
import jax.numpy as jnp
from jax import lax
from jax.experimental import pallas as pl
from jax.experimental.pallas import tpu as pltpu
from jax.experimental.pallas import tpu_sc as plsc

N = 10000
D = 128
E = 320000
NP = 10240
BM = 1000
CHUNK = 64
NW = 32
JPT = 160
NCHUNK_P = NW * JPT
EP = NCHUNK_P * CHUNK
PIECE = JPT // 4
NBUF = 4
RPW = NP // 16

def _wid():
    core = lax.axis_index("c")
    sub = lax.axis_index("s")
    wid = sub * 2 + core
    return core, sub, wid


def _deg_body(dst_hbm, dega_hbm, degb_hbm, didx_v, ones_v, zbuf_v, deg_sh, sem):
    core, sub, wid = _wid()

    @pl.loop(0, CHUNK // 16)
    def _fill_ones(i):
        ones_v[pl.ds(i * 16, 16)] = jnp.full((16,), 1.0, jnp.float32)

    @pl.loop(0, RPW // 16)
    def _fill_zero(i):
        zbuf_v[pl.ds(i * 16, 16)] = jnp.zeros((16,), jnp.float32)

    pltpu.sync_copy(dst_hbm.at[pl.ds(pl.multiple_of(wid * JPT, JPT), JPT)], didx_v)
    pltpu.sync_copy(zbuf_v, deg_sh.at[pl.ds(pl.multiple_of(sub * RPW, RPW), RPW)])
    plsc.subcore_barrier()

    @pl.loop(0, JPT // 8)
    def _count(it):
        descs = [
            pltpu.async_copy(ones_v, deg_sh.at[didx_v.at[it * 8 + b]], sem, add=True)
            for b in range(8)
        ]
        for desc in descs:
            desc.wait()

    plsc.subcore_barrier()
    off = pl.multiple_of(sub * RPW, RPW)

    @pl.when(core == 0)
    def _():
        pltpu.sync_copy(deg_sh.at[pl.ds(off, RPW)], dega_hbm.at[pl.ds(off, RPW)])

    @pl.when(core == 1)
    def _():
        pltpu.sync_copy(deg_sh.at[pl.ds(off, RPW)], degb_hbm.at[pl.ds(off, RPW)])


def _scatter_body(h_hbm, src_hbm, dst_hbm, pa_hbm, pb_hbm,
                  sidx_v, didx_v, rows0_v, rows1_v, rows2_v, rows3_v,
                  acc_sh, gsem0, gsem1, gsem2, gsem3,
                  ssem0, ssem1, ssem2, ssem3):
    core, sub, wid = _wid()
    rows = (rows0_v, rows1_v, rows2_v, rows3_v)
    gsem = (gsem0, gsem1, gsem2, gsem3)
    ssem = (ssem0, ssem1, ssem2, ssem3)

    @pl.loop(0, CHUNK)
    def _zrow(r):
        for cc in range(D // 16):
            rows0_v[r, pl.ds(cc * 16, 16)] = jnp.zeros((16,), jnp.float32)

    for k in range(RPW // CHUNK):
        off = pl.multiple_of(sub * RPW + k * CHUNK, CHUNK)
        pltpu.sync_copy(rows0_v, acc_sh.at[pl.ds(off, CHUNK)])

    plsc.subcore_barrier()

    def wait_gather(rows_v, sem):
        pltpu.make_async_copy(h_hbm.at[sidx_v.at[0]], rows_v, sem).wait()

    def wait_scatter(rows_v, sem):
        pltpu.make_async_copy(rows_v, acc_sh.at[didx_v.at[0]], sem).wait()

    for h in range(JPT // PIECE):
        start = pl.multiple_of(wid * JPT + h * PIECE, PIECE)
        pltpu.sync_copy(src_hbm.at[pl.ds(start, PIECE)], sidx_v)
        pltpu.sync_copy(dst_hbm.at[pl.ds(start, PIECE)], didx_v)

        for b in range(NBUF):
            pltpu.async_copy(h_hbm.at[sidx_v.at[b]], rows[b], gsem[b])

        @pl.loop(0, PIECE // NBUF - 1)
        def _pipe(it):
            j = it * NBUF
            for b in range(NBUF):
                wait_gather(rows[b], gsem[b])
                pltpu.async_copy(rows[b], acc_sh.at[didx_v.at[j + b]], ssem[b],
                                 add=True)
            for b in range(NBUF):
                wait_scatter(rows[b], ssem[b])
                pltpu.async_copy(h_hbm.at[sidx_v.at[j + NBUF + b]], rows[b],
                                 gsem[b])

        for b in range(NBUF):
            wait_gather(rows[b], gsem[b])
            pltpu.async_copy(rows[b], acc_sh.at[didx_v.at[PIECE - NBUF + b]],
                             ssem[b], add=True)
        for b in range(NBUF):
            wait_scatter(rows[b], ssem[b])

    plsc.subcore_barrier()

    woff = pl.multiple_of(sub * RPW, RPW)

    @pl.when(core == 0)
    def _():
        pltpu.sync_copy(acc_sh.at[pl.ds(woff, RPW)], pa_hbm.at[pl.ds(woff, RPW)])

    @pl.when(core == 1)
    def _():
        pltpu.sync_copy(acc_sh.at[pl.ds(woff, RPW)], pb_hbm.at[pl.ds(woff, RPW)])


@functools.lru_cache(maxsize=None)
def _sc_kernels():
    mesh = plsc.VectorSubcoreMesh(
        core_axis_name="c", subcore_axis_name="s", num_cores=2, num_subcores=16
    )
    deg = pl.kernel(
        _deg_body,
        out_type=(
            jax.ShapeDtypeStruct((NP,), jnp.float32),
            jax.ShapeDtypeStruct((NP,), jnp.float32),
        ),
        mesh=mesh,
        scratch_types=[
            pltpu.VMEM((JPT, CHUNK), jnp.int32),
            pltpu.VMEM((CHUNK,), jnp.float32),
            pltpu.VMEM((RPW,), jnp.float32),
            pltpu.VMEM_SHARED((NP,), jnp.float32),
            pltpu.SemaphoreType.DMA,
        ],
    )
    scatter = pl.kernel(
        _scatter_body,
        out_type=(
            jax.ShapeDtypeStruct((NP, D), jnp.float32),
            jax.ShapeDtypeStruct((NP, D), jnp.float32),
        ),
        mesh=mesh,
        scratch_types=[
            pltpu.VMEM((PIECE, CHUNK), jnp.int32),
            pltpu.VMEM((PIECE, CHUNK), jnp.int32),
            pltpu.VMEM((CHUNK, D), jnp.float32),
            pltpu.VMEM((CHUNK, D), jnp.float32),
            pltpu.VMEM((CHUNK, D), jnp.float32),
            pltpu.VMEM((CHUNK, D), jnp.float32),
            pltpu.VMEM_SHARED((NP, D), jnp.float32),
        ] + [pltpu.SemaphoreType.DMA] * 8,
    )
    return deg, scatter


def _t1_body(x_ref, w_ref, dega_ref, degb_ref, h_ref, dis_ref):
    deg = dega_ref[...] + degb_ref[...] + 1.0
    dis = lax.rsqrt(deg)
    dis_ref[...] = dis
    h_ref[...] = (
        jnp.dot(x_ref[...], w_ref[...], preferred_element_type=jnp.float32) * dis
    )


_t1 = pl.pallas_call(
    _t1_body,
    grid=(N // BM,),
    in_specs=[
        pl.BlockSpec((BM, D), lambda i: (i, 0)),
        pl.BlockSpec((D, D), lambda i: (0, 0)),
        pl.BlockSpec((BM, 1), lambda i: (i, 0)),
        pl.BlockSpec((BM, 1), lambda i: (i, 0)),
    ],
    out_specs=[
        pl.BlockSpec((BM, D), lambda i: (i, 0)),
        pl.BlockSpec((BM, 1), lambda i: (i, 0)),
    ],
    out_shape=[
        jax.ShapeDtypeStruct((N, D), jnp.float32),
        jax.ShapeDtypeStruct((N, 1), jnp.float32),
    ],
)


def _pe_table():
    pos = np.arange(N, dtype=np.float32)[:, None]
    div = np.exp(np.arange(0, D, 2, dtype=np.float32) * np.float32(-math.log(10000.0) / D))
    pe = np.zeros((N, D), dtype=np.float32)
    pe[:, 0::2] = np.sin(pos * div, dtype=np.float32)
    pe[:, 1::2] = np.cos(pos * div, dtype=np.float32)
    return pe


_PE = _pe_table()


def _t2_body(pa_ref, pb_ref, h1_ref, dis_ref, pe_ref, b1_ref, w2_ref, h2_ref):
    dis = dis_ref[...]
    agg = dis * (pa_ref[...] + pb_ref[...] + h1_ref[...]) + b1_ref[...]
    x1 = jnp.maximum(agg + pe_ref[...], 0.0)
    h2_ref[...] = (
        jnp.dot(x1, w2_ref[...], preferred_element_type=jnp.float32) * dis
    )


_t2 = pl.pallas_call(
    _t2_body,
    grid=(N // BM,),
    in_specs=[
        pl.BlockSpec((BM, D), lambda i: (i, 0)),
        pl.BlockSpec((BM, D), lambda i: (i, 0)),
        pl.BlockSpec((BM, D), lambda i: (i, 0)),
        pl.BlockSpec((BM, 1), lambda i: (i, 0)),
        pl.BlockSpec((BM, D), lambda i: (i, 0)),
        pl.BlockSpec((1, D), lambda i: (0, 0)),
        pl.BlockSpec((D, D), lambda i: (0, 0)),
    ],
    out_specs=pl.BlockSpec((BM, D), lambda i: (i, 0)),
    out_shape=jax.ShapeDtypeStruct((N, D), jnp.float32),
)


def _t3_body(qa_ref, qb_ref, h2_ref, dis_ref, b2_ref, out_ref):
    out_ref[...] = (
        dis_ref[...] * (qa_ref[...] + qb_ref[...] + h2_ref[...]) + b2_ref[...]
    )


_t3 = pl.pallas_call(
    _t3_body,
    grid=(N // BM,),
    in_specs=[
        pl.BlockSpec((BM, D), lambda i: (i, 0)),
        pl.BlockSpec((BM, D), lambda i: (i, 0)),
        pl.BlockSpec((BM, D), lambda i: (i, 0)),
        pl.BlockSpec((BM, 1), lambda i: (i, 0)),
        pl.BlockSpec((1, D), lambda i: (0, 0)),
    ],
    out_specs=pl.BlockSpec((BM, D), lambda i: (i, 0)),
    out_shape=jax.ShapeDtypeStruct((N, D), jnp.float32),
)


def kernel(basic_block, edge_index, W1, b1, W2, b2):
    ei = edge_index.astype(jnp.int32)
    cyc = jnp.arange(EP - E, dtype=jnp.int32) % (NP - N)
    fill = jnp.stack([cyc, N + cyc])
    srcp, dstp = jnp.concatenate([ei, fill], axis=1).reshape(2, NCHUNK_P, CHUNK)

    _deg_kernel, _scatter_kernel = _sc_kernels()
    dega, degb = _deg_kernel(dstp)
    h1p, dis = _t1(basic_block, W1, dega.reshape(NP, 1), degb.reshape(NP, 1))
    pa, pb = _scatter_kernel(h1p, srcp, dstp)
    h2p = _t2(pa, pb, h1p, dis, jnp.asarray(_PE), b1.reshape(1, D), W2)
    qa, qb = _scatter_kernel(h2p, srcp, dstp)
    return _t3(qa, qb, h2p, dis, b2.reshape(1, D))
